# baseline (speedup 1.0000x reference)
import functools

import jax
import jax.numpy as jnp
from jax import lax
from jax.experimental import pallas as pl
from jax.experimental.pallas import tpu as pltpu
from jax.experimental.pallas import tpu_sc as plsc

N = 10000
D = 256
E = 160000
HALF = D // 2
NC = 2
NS = 16
KC = 128
G = 8
NG = 10
EPT = NG * G * KC
CHD = 40
EPTD = CHD * KC
NPAD = 10240
RPT = NPAD // NS
DEGW = 8
NB = 1000
GRID = N // NB

@functools.cache
def _sc_kernels():
    mesh = plsc.VectorSubcoreMesh(
        core_axis_name="c", subcore_axis_name="s", num_cores=NC, num_subcores=NS
    )

    @functools.partial(
        pl.kernel,
        mesh=mesh,
        out_type=jax.ShapeDtypeStruct((NC, NPAD, DEGW), jnp.float32),
        scratch_types=[
            pltpu.VMEM((CHD, KC), jnp.int32),
            pltpu.VMEM((KC, DEGW), jnp.float32),
            pltpu.VMEM_SHARED((NPAD, DEGW), jnp.float32),
        ],
    )
    def _deg_kernel(dst_hbm, ones_hbm, deg_hbm, dstidx, ones_v, acc):
        c = lax.axis_index("c")
        s = lax.axis_index("s")
        pltpu.sync_copy(ones_hbm, acc.at[pl.ds(s * RPT, RPT)])
        pltpu.sync_copy(ones_hbm.at[pl.ds(0, KC)], ones_v)
        pltpu.sync_copy(dst_hbm.at[c, s], dstidx)
        plsc.subcore_barrier()

        def body(j, carry):
            pltpu.sync_copy(ones_v, acc.at[dstidx.at[j]], add=True)
            return carry

        lax.fori_loop(0, CHD, body, 0)
        plsc.subcore_barrier()
        pltpu.sync_copy(acc.at[pl.ds(s * RPT, RPT)], deg_hbm.at[c, pl.ds(s * RPT, RPT)])

    @functools.partial(
        pl.kernel,
        mesh=mesh,
        out_type=jax.ShapeDtypeStruct((NC, NPAD, HALF), jnp.float32),
        scratch_types=[
            pltpu.VMEM((2, G, KC), jnp.int32),
            pltpu.VMEM((2, G, KC), jnp.int32),
            pltpu.VMEM((2, KC, HALF), jnp.float32),
            pltpu.VMEM_SHARED((NPAD, HALF), jnp.float32),
            pltpu.SemaphoreType.DMA,
            pltpu.SemaphoreType.DMA,
            pltpu.SemaphoreType.DMA,
            pltpu.SemaphoreType.DMA,
        ],
    )
    def _edge_kernel(
        g_hbm, src_hbm, dst_hbm, out_hbm, sidx, didx, rows, acc, semg, semi, sems0, sems1
    ):
        c = lax.axis_index("c")
        s = lax.axis_index("s")
        pltpu.sync_copy(g_hbm.at[c, pl.ds(s * RPT, RPT)], acc.at[pl.ds(s * RPT, RPT)])
        pltpu.sync_copy(src_hbm.at[s, 0], sidx.at[0])
        pltpu.sync_copy(dst_hbm.at[s, 0], didx.at[0])
        plsc.subcore_barrier()

        gtab = g_hbm.at[c]
        pltpu.async_copy(src_hbm.at[s, 1], sidx.at[1], semi)
        pltpu.async_copy(dst_hbm.at[s, 1], didx.at[1], semi)
        pltpu.async_copy(gtab.at[sidx.at[0, 0, pl.ds(0, 64)]], rows.at[0, pl.ds(0, 64)], semg)
        pltpu.async_copy(gtab.at[sidx.at[0, 0, pl.ds(64, 64)]], rows.at[0, pl.ds(64, 64)], semg)

        def subg(gb, k, h, b):
            return pltpu.async_copy(
                gtab.at[sidx.at[gb, k, pl.ds(h * 64, 64)]],
                rows.at[b, pl.ds(h * 64, 64)],
                semg,
            )

        def wait_subg(gb, k, h, b):
            pltpu.make_async_copy(
                gtab.at[sidx.at[gb, k, pl.ds(h * 64, 64)]],
                rows.at[b, pl.ds(h * 64, 64)],
                semg,
            ).wait()

        def pair(p, carry):
            for gb in (0, 1):
                grp = 2 * p + gb
                for k in range(G):
                    b = k % 2
                    sems = (sems0, sems1)[b]
                    semo = (sems0, sems1)[1 - b]
                    wait_subg(gb, k, 0, b)
                    wait_subg(gb, k, 1, b)
                    pltpu.async_copy(rows.at[b], acc.at[didx.at[gb, k]], sems, add=True)
                    j = grp * G + k

                    @pl.when(j > 0)
                    def _():
                        pltpu.make_async_copy(
                            rows.at[1 - b], acc.at[didx.at[gb, k]], semo
                        ).wait()

                    if k < G - 1:
                        subg(gb, k + 1, 0, 1 - b)
                        subg(gb, k + 1, 1, 1 - b)
                    else:

                        @pl.when(grp + 1 < NG)
                        def _():
                            pltpu.make_async_copy(
                                src_hbm.at[s, grp + 1], sidx.at[1 - gb], semi
                            ).wait()
                            pltpu.make_async_copy(
                                dst_hbm.at[s, grp + 1], didx.at[1 - gb], semi
                            ).wait()
                            subg(1 - gb, 0, 0, 1 - b)
                            subg(1 - gb, 0, 1, 1 - b)

                        @pl.when(grp + 2 < NG)
                        def _():
                            pltpu.async_copy(
                                src_hbm.at[s, grp + 2], sidx.at[gb], semi
                            )
                            pltpu.async_copy(
                                dst_hbm.at[s, grp + 2], didx.at[gb], semi
                            )
            return carry

        lax.fori_loop(0, NG // 2, pair, 0)
        pltpu.make_async_copy(
            rows.at[(G - 1) % 2], acc.at[didx.at[1, G - 1]], (sems0, sems1)[(G - 1) % 2]
        ).wait()
        plsc.subcore_barrier()
        pltpu.sync_copy(acc.at[pl.ds(s * RPT, RPT)], out_hbm.at[c, pl.ds(s * RPT, RPT)])

    return _deg_kernel, _edge_kernel


def _dinv_block(degp):
    return lax.rsqrt(degp[0, :, 0:1] + degp[1, :, 0:1] - 1.0)


def _t1_body(x_ref, w_ref, degq_ref, g_ref):
    dinv = _dinv_block(degq_ref[...])
    h = jnp.dot(
        x_ref[...].astype(jnp.bfloat16),
        w_ref[...].astype(jnp.bfloat16),
        preferred_element_type=jnp.float32,
    )
    g = h * dinv
    g_ref[0] = g[:, :HALF]
    g_ref[1] = g[:, HALF:]


def _t2_body(acc_ref, w_ref, b1_ref, degq_ref, g_ref):
    dinv = _dinv_block(degq_ref[...])
    b1 = b1_ref[...]
    o0 = jnp.maximum(acc_ref[0] * dinv + b1[:, :HALF], 0.0)
    o1 = jnp.maximum(acc_ref[1] * dinv + b1[:, HALF:], 0.0)
    o = jnp.concatenate([o0, o1], axis=1).astype(jnp.bfloat16)
    h = jnp.dot(
        o, w_ref[...].astype(jnp.bfloat16), preferred_element_type=jnp.float32
    )
    g = h * dinv
    g_ref[0] = g[:, :HALF]
    g_ref[1] = g[:, HALF:]


def _t3_body(acc_ref, b2_ref, degq_ref, out_ref):
    dinv = _dinv_block(degq_ref[...])
    b2 = b2_ref[...]
    out_ref[:, :HALF] = jnp.maximum(2.0 * (acc_ref[0] * dinv + b2[:, :HALF]), 0.0)
    out_ref[:, HALF:] = jnp.maximum(2.0 * (acc_ref[1] * dinv + b2[:, HALF:]), 0.0)


_HALVES_SPEC = pl.BlockSpec((NC, NB, HALF), lambda i: (0, i, 0))
_DEGQ_SPEC = pl.BlockSpec((NC, NB, DEGW), lambda i: (0, i, 0))
_W_SPEC = pl.BlockSpec((D, D), lambda i: (0, 0))
_B_SPEC = pl.BlockSpec((1, D), lambda i: (0, 0))
_HALVES_TY = jax.ShapeDtypeStruct((NC, NPAD, HALF), jnp.float32)

_t1 = pl.pallas_call(
    _t1_body,
    grid=(GRID,),
    in_specs=[pl.BlockSpec((NB, D), lambda i: (i, 0)), _W_SPEC, _DEGQ_SPEC],
    out_specs=_HALVES_SPEC,
    out_shape=_HALVES_TY,
)

_t2 = pl.pallas_call(
    _t2_body,
    grid=(GRID,),
    in_specs=[_HALVES_SPEC, _W_SPEC, _B_SPEC, _DEGQ_SPEC],
    out_specs=_HALVES_SPEC,
    out_shape=_HALVES_TY,
)

_t3 = pl.pallas_call(
    _t3_body,
    grid=(GRID,),
    in_specs=[_HALVES_SPEC, _B_SPEC, _DEGQ_SPEC],
    out_specs=pl.BlockSpec((NB, D), lambda i: (i, 0)),
    out_shape=jax.ShapeDtypeStruct((N, D), jnp.float32),
)


def kernel(x, edge_index, W1, b1, W2, b2):
    src = edge_index[0]
    dst = edge_index[1]
    ept0 = E // NS
    srcp = jnp.pad(src.reshape(NS, ept0), ((0, 0), (0, EPT - ept0)))
    dstp = jnp.pad(
        dst.reshape(NS, ept0), ((0, 0), (0, EPT - ept0)), constant_values=NPAD - 1
    )
    src_e = srcp.reshape(NS, NG, G, KC)
    dst_e = dstp.reshape(NS, NG, G, KC)
    dst_k1 = dstp.reshape(NC, NS, CHD, KC)
    ones = jnp.ones((RPT, DEGW), jnp.float32)
    b1r = b1.reshape(1, D)
    b2r = b2.reshape(1, D)

    _deg_kernel, _edge_kernel = _sc_kernels()
    degq = _deg_kernel(dst_k1, ones)

    g1 = _t1(x, W1, degq)
    acc1 = _edge_kernel(g1, src_e, dst_e)
    g2 = _t2(acc1, W2, b1r, degq)
    acc2 = _edge_kernel(g2, src_e, dst_e)
    return _t3(acc2, b2r, degq)


# speedup vs baseline: 1.0054x; 1.0054x over previous
import functools

import jax
import jax.numpy as jnp
from jax import lax
from jax.experimental import pallas as pl
from jax.experimental.pallas import tpu as pltpu
from jax.experimental.pallas import tpu_sc as plsc

N = 10000
D = 256
E = 160000
HALF = D // 2
NC = 2
NS = 16
KC = 128
G = 8
NG = 10
EPT = NG * G * KC
CHD = 40
EPTD = CHD * KC
NPAD = 10240
RPT = NPAD // NS
DEGW = 8
NB = 1000
GRID = N // NB

@functools.cache
def _sc_kernels():
    mesh = plsc.VectorSubcoreMesh(
        core_axis_name="c", subcore_axis_name="s", num_cores=NC, num_subcores=NS
    )

    @functools.partial(
        pl.kernel,
        mesh=mesh,
        out_type=jax.ShapeDtypeStruct((NC, NPAD, DEGW), jnp.float32),
        scratch_types=[
            pltpu.VMEM((CHD, KC), jnp.int32),
            pltpu.VMEM((KC, DEGW), jnp.float32),
            pltpu.VMEM_SHARED((NPAD, DEGW), jnp.float32),
        ],
    )
    def _deg_kernel(dst_hbm, ones_hbm, deg_hbm, dstidx, ones_v, acc):
        c = lax.axis_index("c")
        s = lax.axis_index("s")
        pltpu.sync_copy(ones_hbm.at[pl.ds(s * RPT, RPT)], acc.at[pl.ds(s * RPT, RPT)])
        pltpu.sync_copy(ones_hbm.at[pl.ds(0, KC)], ones_v)
        pltpu.sync_copy(dst_hbm.at[c, s], dstidx)
        plsc.subcore_barrier()

        def body(j, carry):
            pltpu.sync_copy(ones_v, acc.at[dstidx.at[j]], add=True)
            return carry

        lax.fori_loop(0, CHD, body, 0)
        plsc.subcore_barrier()
        pltpu.sync_copy(acc.at[pl.ds(s * RPT, RPT)], deg_hbm.at[c, pl.ds(s * RPT, RPT)])

    @functools.partial(
        pl.kernel,
        mesh=mesh,
        out_type=jax.ShapeDtypeStruct((NC, NPAD, HALF), jnp.float32),
        scratch_types=[
            pltpu.VMEM((2, G, KC), jnp.int32),
            pltpu.VMEM((2, G, KC), jnp.int32),
            pltpu.VMEM((2, KC, HALF), jnp.float32),
            pltpu.VMEM_SHARED((NPAD, HALF), jnp.float32),
            pltpu.SemaphoreType.DMA,
            pltpu.SemaphoreType.DMA,
            pltpu.SemaphoreType.DMA,
            pltpu.SemaphoreType.DMA,
        ],
    )
    def _edge_kernel(
        g_hbm, src_hbm, dst_hbm, out_hbm, sidx, didx, rows, acc, semg, semi, sems0, sems1
    ):
        c = lax.axis_index("c")
        s = lax.axis_index("s")
        pltpu.sync_copy(g_hbm.at[c, pl.ds(s * RPT, RPT)], acc.at[pl.ds(s * RPT, RPT)])
        pltpu.sync_copy(src_hbm.at[s, 0], sidx.at[0])
        pltpu.sync_copy(dst_hbm.at[s, 0], didx.at[0])
        plsc.subcore_barrier()

        gtab = g_hbm.at[c]
        pltpu.async_copy(src_hbm.at[s, 1], sidx.at[1], semi)
        pltpu.async_copy(dst_hbm.at[s, 1], didx.at[1], semi)
        pltpu.async_copy(gtab.at[sidx.at[0, 0, pl.ds(0, 64)]], rows.at[0, pl.ds(0, 64)], semg)
        pltpu.async_copy(gtab.at[sidx.at[0, 0, pl.ds(64, 64)]], rows.at[0, pl.ds(64, 64)], semg)

        def subg(gb, k, h, b):
            return pltpu.async_copy(
                gtab.at[sidx.at[gb, k, pl.ds(h * 64, 64)]],
                rows.at[b, pl.ds(h * 64, 64)],
                semg,
            )

        def wait_subg(gb, k, h, b):
            pltpu.make_async_copy(
                gtab.at[sidx.at[gb, k, pl.ds(h * 64, 64)]],
                rows.at[b, pl.ds(h * 64, 64)],
                semg,
            ).wait()

        def pair(p, carry):
            for gb in (0, 1):
                grp = 2 * p + gb
                for k in range(G):
                    b = k % 2
                    sems = (sems0, sems1)[b]
                    semo = (sems0, sems1)[1 - b]
                    wait_subg(gb, k, 0, b)
                    wait_subg(gb, k, 1, b)
                    pltpu.async_copy(rows.at[b], acc.at[didx.at[gb, k]], sems, add=True)
                    j = grp * G + k

                    @pl.when(j > 0)
                    def _():
                        pltpu.make_async_copy(
                            rows.at[1 - b], acc.at[didx.at[gb, k]], semo
                        ).wait()

                    if k < G - 1:
                        subg(gb, k + 1, 0, 1 - b)
                        subg(gb, k + 1, 1, 1 - b)
                    else:

                        @pl.when(grp + 1 < NG)
                        def _():
                            pltpu.make_async_copy(
                                src_hbm.at[s, grp + 1], sidx.at[1 - gb], semi
                            ).wait()
                            pltpu.make_async_copy(
                                dst_hbm.at[s, grp + 1], didx.at[1 - gb], semi
                            ).wait()
                            subg(1 - gb, 0, 0, 1 - b)
                            subg(1 - gb, 0, 1, 1 - b)

                        @pl.when(grp + 2 < NG)
                        def _():
                            pltpu.async_copy(
                                src_hbm.at[s, grp + 2], sidx.at[gb], semi
                            )
                            pltpu.async_copy(
                                dst_hbm.at[s, grp + 2], didx.at[gb], semi
                            )
            return carry

        lax.fori_loop(0, NG // 2, pair, 0)
        pltpu.make_async_copy(
            rows.at[(G - 1) % 2], acc.at[didx.at[1, G - 1]], (sems0, sems1)[(G - 1) % 2]
        ).wait()
        plsc.subcore_barrier()
        pltpu.sync_copy(acc.at[pl.ds(s * RPT, RPT)], out_hbm.at[c, pl.ds(s * RPT, RPT)])

    return _deg_kernel, _edge_kernel


def _dinv_block(degp):
    return lax.rsqrt(degp[0, :, 0:1] + degp[1, :, 0:1] - 1.0)


def _t1_body(x_ref, w_ref, degq_ref, g_ref):
    dinv = _dinv_block(degq_ref[...])
    h = jnp.dot(
        x_ref[...].astype(jnp.bfloat16),
        w_ref[...].astype(jnp.bfloat16),
        preferred_element_type=jnp.float32,
    )
    g = h * dinv
    g_ref[0] = g[:, :HALF]
    g_ref[1] = g[:, HALF:]


def _t2_body(acc_ref, w_ref, b1_ref, degq_ref, g_ref):
    dinv = _dinv_block(degq_ref[...])
    b1 = b1_ref[...]
    o0 = jnp.maximum(acc_ref[0] * dinv + b1[:, :HALF], 0.0)
    o1 = jnp.maximum(acc_ref[1] * dinv + b1[:, HALF:], 0.0)
    o = jnp.concatenate([o0, o1], axis=1).astype(jnp.bfloat16)
    h = jnp.dot(
        o, w_ref[...].astype(jnp.bfloat16), preferred_element_type=jnp.float32
    )
    g = h * dinv
    g_ref[0] = g[:, :HALF]
    g_ref[1] = g[:, HALF:]


def _t3_body(acc_ref, b2_ref, degq_ref, out_ref):
    dinv = _dinv_block(degq_ref[...])
    b2 = b2_ref[...]
    out_ref[:, :HALF] = jnp.maximum(2.0 * (acc_ref[0] * dinv + b2[:, :HALF]), 0.0)
    out_ref[:, HALF:] = jnp.maximum(2.0 * (acc_ref[1] * dinv + b2[:, HALF:]), 0.0)


_HALVES_SPEC = pl.BlockSpec((NC, NB, HALF), lambda i: (0, i, 0))
_DEGQ_SPEC = pl.BlockSpec((NC, NB, DEGW), lambda i: (0, i, 0))
_W_SPEC = pl.BlockSpec((D, D), lambda i: (0, 0))
_B_SPEC = pl.BlockSpec((1, D), lambda i: (0, 0))
_HALVES_TY = jax.ShapeDtypeStruct((NC, NPAD, HALF), jnp.float32)

_t1 = pl.pallas_call(
    _t1_body,
    grid=(GRID,),
    in_specs=[pl.BlockSpec((NB, D), lambda i: (i, 0)), _W_SPEC, _DEGQ_SPEC],
    out_specs=_HALVES_SPEC,
    out_shape=_HALVES_TY,
)

_t2 = pl.pallas_call(
    _t2_body,
    grid=(GRID,),
    in_specs=[_HALVES_SPEC, _W_SPEC, _B_SPEC, _DEGQ_SPEC],
    out_specs=_HALVES_SPEC,
    out_shape=_HALVES_TY,
)

_t3 = pl.pallas_call(
    _t3_body,
    grid=(GRID,),
    in_specs=[_HALVES_SPEC, _B_SPEC, _DEGQ_SPEC],
    out_specs=pl.BlockSpec((NB, D), lambda i: (i, 0)),
    out_shape=jax.ShapeDtypeStruct((N, D), jnp.float32),
)


def kernel(x, edge_index, W1, b1, W2, b2):
    src = edge_index[0]
    dst = edge_index[1]
    ept0 = E // NS
    srcp = jnp.pad(src.reshape(NS, ept0), ((0, 0), (0, EPT - ept0)))
    dstp = jnp.pad(
        dst.reshape(NS, ept0), ((0, 0), (0, EPT - ept0)), constant_values=NPAD - 1
    )
    src_e = srcp.reshape(NS, NG, G, KC)
    dst_e = dstp.reshape(NS, NG, G, KC)
    eptd0 = E // (NC * NS)
    dst_k1 = jnp.pad(
        dst.reshape(NC * NS, eptd0),
        ((0, 0), (0, EPTD - eptd0)),
        constant_values=NPAD - 1,
    ).reshape(NC, NS, CHD, KC)
    ones = jnp.ones((NPAD, DEGW), jnp.float32)
    b1r = b1.reshape(1, D)
    b2r = b2.reshape(1, D)

    _deg_kernel, _edge_kernel = _sc_kernels()
    degq = _deg_kernel(dst_k1, ones)

    g1 = _t1(x, W1, degq)
    acc1 = _edge_kernel(g1, src_e, dst_e)
    g2 = _t2(acc1, W2, b1r, degq)
    acc2 = _edge_kernel(g2, src_e, dst_e)
    return _t3(acc2, b2r, degq)
